# baseline (speedup 1.0000x reference)
import jax
import jax.numpy as jnp
from jax.experimental import pallas as pl

_PRE_NMS_THRESH = 0.05
_PRE_NMS_TOP_N = 1000
_POST_TOP_N = 100
_NMS_IOU = 0.5


def _sig_body(x_ref, o_ref):
    o_ref[...] = jax.nn.sigmoid(x_ref[...])


def _pallas_sigmoid(x):
    return pl.pallas_call(
        _sig_body,
        out_shape=jax.ShapeDtypeStruct(x.shape, x.dtype),
    )(x)


def _nms_keep(boxes, scores, valid):
    K = boxes.shape[0]
    s = jnp.where(valid, scores, -jnp.inf)
    order = jnp.argsort(-s)
    b = boxes[order]
    areas = (b[:, 2] - b[:, 0] + 1.0) * (b[:, 3] - b[:, 1] + 1.0)
    idx = jnp.arange(K)

    def body(i, keep):
        xx1 = jnp.maximum(b[i, 0], b[:, 0])
        yy1 = jnp.maximum(b[i, 1], b[:, 1])
        xx2 = jnp.minimum(b[i, 2], b[:, 2])
        yy2 = jnp.minimum(b[i, 3], b[:, 3])
        inter = jnp.maximum(xx2 - xx1 + 1.0, 0.0) * jnp.maximum(yy2 - yy1 + 1.0, 0.0)
        iou = inter / (areas[i] + areas - inter)
        suppress = (iou > _NMS_IOU) & (idx > i) & keep[i]
        return keep & (~suppress)

    keep_sorted = jax.lax.fori_loop(0, K, body, jnp.ones((K,), dtype=bool))
    keep = jnp.zeros((K,), dtype=bool).at[order].set(keep_sorted)
    return keep


def kernel(locations, box_cls, box_regression, centerness, lof_tag, image_sizes):
    N, C, H, W = box_cls.shape
    cls_sig = _pallas_sigmoid(box_cls)
    ctr_sig = _pallas_sigmoid(centerness)
    cls = jnp.transpose(cls_sig, (0, 2, 3, 1)).reshape(N, -1, C)
    reg = jnp.transpose(box_regression, (0, 2, 3, 1)).reshape(N, -1, 4)
    ctr = jnp.transpose(ctr_sig, (0, 2, 3, 1)).reshape(N, -1)
    tag = jnp.round(jnp.transpose(lof_tag, (0, 2, 3, 1)).reshape(N, -1)).astype(jnp.int32)
    scores_max = jnp.max(cls, axis=-1)
    labels = jnp.argmax(cls, axis=-1) + 1
    cand = scores_max > _PRE_NMS_THRESH
    scores = scores_max * ctr
    masked = jnp.where(cand, scores, -1.0)

    out_boxes, out_scores, out_labels, out_tags = [], [], [], []
    for i in range(N):
        topv, topi = jax.lax.top_k(masked[i], _PRE_NMS_TOP_N)
        loc_i = locations[topi]
        reg_i = reg[i][topi]
        lab_i = labels[i][topi]
        tag_i = tag[i][topi]
        det = jnp.stack([loc_i[:, 0] - reg_i[:, 0],
                         loc_i[:, 1] - reg_i[:, 1],
                         loc_i[:, 0] + reg_i[:, 2],
                         loc_i[:, 1] + reg_i[:, 3]], axis=1)
        h = image_sizes[i, 0].astype(jnp.float32)
        w = image_sizes[i, 1].astype(jnp.float32)
        det = jnp.stack([jnp.clip(det[:, 0], 0.0, w - 1.0),
                         jnp.clip(det[:, 1], 0.0, h - 1.0),
                         jnp.clip(det[:, 2], 0.0, w - 1.0),
                         jnp.clip(det[:, 3], 0.0, h - 1.0)], axis=1)
        ws = det[:, 2] - det[:, 0] + 1.0
        hs = det[:, 3] - det[:, 1] + 1.0
        valid = (topv > 0.0) & (ws >= 0) & (hs >= 0)
        off = lab_i.astype(jnp.float32) * 100000.0 + tag_i.astype(jnp.float32) * 10000.0
        keep = _nms_keep(jax.lax.stop_gradient(det + off[:, None]),
                         jax.lax.stop_gradient(topv), valid)
        final = jnp.where(keep & valid, topv, -1.0)
        fv, fi = jax.lax.top_k(final, _POST_TOP_N)
        out_boxes.append(det[fi])
        out_scores.append(fv)
        out_labels.append(lab_i[fi])
        out_tags.append(tag_i[fi])
    return (jnp.stack(out_boxes), jnp.stack(out_scores),
            jnp.stack(out_labels), jnp.stack(out_tags))


# speedup vs baseline: 21.1424x; 21.1424x over previous
import jax
import jax.numpy as jnp
from jax import lax
from jax.experimental import pallas as pl

_PRE_NMS_THRESH = 0.05
_PRE_NMS_TOP_N = 1000
_POST_TOP_N = 100
_NMS_IOU = 0.5
_PAD = 1024


def _score_body(cls_ref, ctr_ref, tag_ref, masked_ref, lab_ref, tago_ref):
    x = cls_ref[0]
    sig = jax.nn.sigmoid(x)
    smax = jnp.max(sig, axis=0, keepdims=True)
    c_iota = lax.broadcasted_iota(jnp.int32, x.shape, 0)
    is_max = sig == smax
    lab = jnp.min(jnp.where(is_max, c_iota, x.shape[0]), axis=0, keepdims=True)
    ctr = jax.nn.sigmoid(ctr_ref[0])
    scores = smax * ctr
    masked_ref[0] = jnp.where(smax > _PRE_NMS_THRESH, scores, -1.0)
    lab_ref[0] = lab + 1
    tago_ref[0] = jnp.round(tag_ref[0]).astype(jnp.int32)


def _score_stage(cls3, ctr3, tag3):
    N, C, HW = cls3.shape
    grid = (N,)
    in_specs = [
        pl.BlockSpec((1, C, HW), lambda n: (n, 0, 0)),
        pl.BlockSpec((1, 1, HW), lambda n: (n, 0, 0)),
        pl.BlockSpec((1, 1, HW), lambda n: (n, 0, 0)),
    ]
    out_specs = [
        pl.BlockSpec((1, 1, HW), lambda n: (n, 0, 0)),
        pl.BlockSpec((1, 1, HW), lambda n: (n, 0, 0)),
        pl.BlockSpec((1, 1, HW), lambda n: (n, 0, 0)),
    ]
    out_shape = [
        jax.ShapeDtypeStruct((N, 1, HW), jnp.float32),
        jax.ShapeDtypeStruct((N, 1, HW), jnp.int32),
        jax.ShapeDtypeStruct((N, 1, HW), jnp.int32),
    ]
    masked, lab, tag = pl.pallas_call(
        _score_body, grid=grid, in_specs=in_specs, out_specs=out_specs,
        out_shape=out_shape,
    )(cls3, ctr3, tag3)
    return masked.reshape(N, HW), lab.reshape(N, HW), tag.reshape(N, HW)


def _nms_body(tv_ref, lx_ref, ly_ref, r0_ref, r1_ref, r2_ref, r3_ref,
              lab_ref, tag_ref, wm1_ref, hm1_ref,
              fin_ref, d1_ref, d2_ref, d3_ref, d4_ref):
    N = tv_ref.shape[0]
    for n in range(N):
        tv = tv_ref[n]
        lx, ly = lx_ref[n], ly_ref[n]
        wm1, hm1 = wm1_ref[n], hm1_ref[n]
        zero = jnp.float32(0.0)
        x1 = jnp.minimum(jnp.maximum(lx - r0_ref[n], zero), wm1)
        y1 = jnp.minimum(jnp.maximum(ly - r1_ref[n], zero), hm1)
        x2 = jnp.minimum(jnp.maximum(lx + r2_ref[n], zero), wm1)
        y2 = jnp.minimum(jnp.maximum(ly + r3_ref[n], zero), hm1)
        ws = x2 - x1 + 1.0
        hs = y2 - y1 + 1.0
        valid = (tv > 0.0) & (ws >= 0.0) & (hs >= 0.0)
        validf = jnp.where(valid, 1.0, 0.0)
        off = lab_ref[n] * 100000.0 + tag_ref[n] * 10000.0
        bx1, by1 = x1 + off, y1 + off
        bx2, by2 = x2 + off, y2 + off
        areas = (bx2 - bx1 + 1.0) * (by2 - by1 + 1.0)
        row = lax.broadcasted_iota(jnp.int32, (8, 128), 0)
        col = lax.broadcasted_iota(jnp.int32, (8, 128), 1)
        idx = row * 128 + col
        neg = jnp.float32(-3.0e38)

        def body(i, keep_f):
            sel = idx == i
            xi1 = jnp.max(jnp.where(sel, bx1, neg))
            yi1 = jnp.max(jnp.where(sel, by1, neg))
            xi2 = jnp.max(jnp.where(sel, bx2, neg))
            yi2 = jnp.max(jnp.where(sel, by2, neg))
            ai = jnp.max(jnp.where(sel, areas, neg))
            ki = jnp.max(jnp.where(sel, keep_f * validf, 0.0))
            iw = jnp.maximum(jnp.minimum(xi2, bx2) - jnp.maximum(xi1, bx1) + 1.0, 0.0)
            ih = jnp.maximum(jnp.minimum(yi2, by2) - jnp.maximum(yi1, by1) + 1.0, 0.0)
            inter = iw * ih
            iou = inter / (ai + areas - inter)
            sup = jnp.where((iou > _NMS_IOU) & (idx > i), 1.0, 0.0)
            return keep_f * (1.0 - sup * ki)

        keep_f = lax.fori_loop(0, _PRE_NMS_TOP_N, body,
                               jnp.ones((8, 128), dtype=jnp.float32))
        fin_ref[n] = jnp.where((keep_f > 0.0) & valid, tv, -1.0)
        d1_ref[n] = x1
        d2_ref[n] = y1
        d3_ref[n] = x2
        d4_ref[n] = y2


def _nms_stage(tv, lx, ly, r0, r1, r2, r3, labf, tagf, wm1, hm1):
    N = tv.shape[0]
    out_shape = [jax.ShapeDtypeStruct((N, 8, 128), jnp.float32)] * 5
    return pl.pallas_call(_nms_body, out_shape=out_shape)(
        tv, lx, ly, r0, r1, r2, r3, labf, tagf, wm1, hm1)


def _pad_tile(x, fill):
    N, K = x.shape
    return jnp.pad(x, ((0, 0), (0, _PAD - K)), constant_values=fill).reshape(N, 8, 128)


def kernel(locations, box_cls, box_regression, centerness, lof_tag, image_sizes):
    N, C, H, W = box_cls.shape
    HW = H * W
    masked, labels, tags = _score_stage(
        box_cls.reshape(N, C, HW),
        centerness.reshape(N, 1, HW),
        lof_tag.reshape(N, 1, HW),
    )
    topv, topi = lax.top_k(masked, _PRE_NMS_TOP_N)
    loc_g = jnp.take(locations, topi, axis=0)
    regr = box_regression.reshape(N, 4, HW)
    reg_g = jnp.take_along_axis(regr, topi[:, None, :], axis=2)
    lab_g = jnp.take_along_axis(labels, topi, axis=1)
    tag_g = jnp.take_along_axis(tags, topi, axis=1)

    img = image_sizes.astype(jnp.float32)
    wm1 = jnp.broadcast_to((img[:, 1] - 1.0)[:, None], (N, _PAD)).reshape(N, 8, 128)
    hm1 = jnp.broadcast_to((img[:, 0] - 1.0)[:, None], (N, _PAD)).reshape(N, 8, 128)

    fin, d1, d2, d3, d4 = _nms_stage(
        _pad_tile(topv, -1.0),
        _pad_tile(loc_g[:, :, 0], 0.0), _pad_tile(loc_g[:, :, 1], 0.0),
        _pad_tile(reg_g[:, 0, :], 0.0), _pad_tile(reg_g[:, 1, :], 0.0),
        _pad_tile(reg_g[:, 2, :], 0.0), _pad_tile(reg_g[:, 3, :], 0.0),
        _pad_tile(lab_g.astype(jnp.float32), 0.0),
        _pad_tile(tag_g.astype(jnp.float32), 0.0),
        wm1, hm1,
    )
    final = fin.reshape(N, _PAD)[:, :_PRE_NMS_TOP_N]
    det = jnp.stack([d1, d2, d3, d4], axis=-1).reshape(N, _PAD, 4)[:, :_PRE_NMS_TOP_N]
    fv, fi = lax.top_k(final, _POST_TOP_N)
    out_boxes = jnp.take_along_axis(det, fi[:, :, None], axis=1)
    out_labels = jnp.take_along_axis(lab_g, fi, axis=1)
    out_tags = jnp.take_along_axis(tag_g, fi, axis=1)
    return out_boxes, fv, out_labels, out_tags
